# baseline (speedup 1.0000x reference)
import jax
import jax.numpy as jnp
from jax import lax
from jax.experimental import pallas as pl
from jax.experimental.pallas import tpu as pltpu
from jax.experimental.pallas import tpu_sc as plsc

VOCAB = 1000000
EMBED_DIM = 64
BATCH = 4096
HIST = 200

_INFO = plsc.get_sparse_core_info()
NC, NS, L = _INFO.num_cores, _INFO.num_subcores, _INFO.num_lanes
NW = NC * NS

B = BATCH * HIST
B_PER_W = B // NW
IDX_MINOR = 128
K = 8
CHUNK = K * IDX_MINOR
N_CHUNKS = B_PER_W // CHUNK
ROWS_PER_W = B_PER_W // IDX_MINOR


def _gather_body(x_hbm, table_hbm, out_hbm, idx_v, rows_v, gsem):
    wid = lax.axis_index("s") * NC + lax.axis_index("c")
    base = wid * B_PER_W
    idx_row_base = wid * ROWS_PER_W

    def chunk(g, carry):
        off = base + g * CHUNK
        pltpu.sync_copy(x_hbm.at[pl.ds(idx_row_base + g * K, K)], idx_v)
        copies = []
        for j in range(K):
            copies.append(pltpu.async_copy(
                table_hbm.at[idx_v.at[j]],
                rows_v.at[pl.ds(j * IDX_MINOR, IDX_MINOR)],
                gsem))
        for c in copies:
            c.wait()
        pltpu.sync_copy(rows_v, out_hbm.at[pl.ds(off, CHUNK)])
        return carry

    lax.fori_loop(0, N_CHUNKS, chunk, 0)


def kernel(x, table):
    x2 = x.reshape(B // IDX_MINOR, IDX_MINOR).astype(jnp.int32)
    mesh = plsc.VectorSubcoreMesh(core_axis_name="c", subcore_axis_name="s")
    gathered = pl.kernel(
        _gather_body,
        mesh=mesh,
        out_type=jax.ShapeDtypeStruct((B, EMBED_DIM), jnp.float32),
        scratch_types=[
            pltpu.VMEM((K, IDX_MINOR), jnp.int32),
            pltpu.VMEM((CHUNK, EMBED_DIM), jnp.float32),
            pltpu.SemaphoreType.DMA,
        ],
        compiler_params=pltpu.CompilerParams(use_tc_tiling_on_sc=False),
    )(x2, table)
    return gathered.reshape(BATCH, HIST, EMBED_DIM)


# speedup vs baseline: 1.0170x; 1.0170x over previous
import jax
import jax.numpy as jnp
from jax import lax
from jax.experimental import pallas as pl
from jax.experimental.pallas import tpu as pltpu
from jax.experimental.pallas import tpu_sc as plsc

VOCAB = 1000000
EMBED_DIM = 64
BATCH = 4096
HIST = 200

_INFO = plsc.get_sparse_core_info()
NC, NS, L = _INFO.num_cores, _INFO.num_subcores, _INFO.num_lanes
NW = NC * NS

B = BATCH * HIST
B_PER_W = B // NW
CHUNK = 256
N_CHUNKS = B_PER_W // CHUNK
NBUF = 4
LOOKAHEAD = 2


def _gather_body(x_hbm, table_hbm, out_hbm, idx_v, rows_v,
                 g0, g1, g2, g3, o0, o1, o2, o3):
    gsems = (g0, g1, g2, g3)
    osems = (o0, o1, o2, o3)
    wid = lax.axis_index("s") * NC + lax.axis_index("c")
    base = wid * B_PER_W
    pltpu.sync_copy(x_hbm.at[pl.ds(base, B_PER_W)], idx_v)

    def fire_gather(c, b):
        pltpu.async_copy(
            table_hbm.at[idx_v.at[pl.ds(c * CHUNK, CHUNK)]],
            rows_v.at[b], gsems[b])

    def wait_gather(c, b):
        pltpu.make_async_copy(
            table_hbm.at[idx_v.at[pl.ds(c * CHUNK, CHUNK)]],
            rows_v.at[b], gsems[b]).wait()

    def fire_out(c, b):
        pltpu.async_copy(
            rows_v.at[b], out_hbm.at[pl.ds(base + c * CHUNK, CHUNK)], osems[b])

    def wait_out(c, b):
        pltpu.make_async_copy(
            rows_v.at[b], out_hbm.at[pl.ds(base + c * CHUNK, CHUNK)],
            osems[b]).wait()

    for c in range(LOOKAHEAD):
        fire_gather(c, c % NBUF)

    def round_body(r, carry):
        for b in range(NBUF):
            c = r * NBUF + b
            c2 = c + LOOKAHEAD
            b2 = (b + LOOKAHEAD) % NBUF

            @pl.when(c2 < N_CHUNKS)
            def _():
                @pl.when(c2 >= NBUF)
                def _():
                    wait_out(c - LOOKAHEAD, b2)
                fire_gather(c2, b2)

            wait_gather(c, b)
            fire_out(c, b)
        return carry

    lax.fori_loop(0, N_CHUNKS // NBUF, round_body, 0)

    for k in range(NBUF):
        c = N_CHUNKS - NBUF + k
        wait_out(c, c % NBUF)


def kernel(x, table):
    x1 = x.reshape(B).astype(jnp.int32)
    mesh = plsc.VectorSubcoreMesh(core_axis_name="c", subcore_axis_name="s")
    gathered = pl.kernel(
        _gather_body,
        mesh=mesh,
        out_type=jax.ShapeDtypeStruct((B, EMBED_DIM), jnp.float32),
        scratch_types=[
            pltpu.VMEM((B_PER_W,), jnp.int32),
            pltpu.VMEM((NBUF, CHUNK, EMBED_DIM), jnp.float32),
        ] + [pltpu.SemaphoreType.DMA] * (2 * NBUF),
        compiler_params=pltpu.CompilerParams(use_tc_tiling_on_sc=False),
    )(x1, table)
    return gathered.reshape(BATCH, HIST, EMBED_DIM)


# speedup vs baseline: 1.0176x; 1.0007x over previous
import jax
import jax.numpy as jnp
from jax import lax
from jax.experimental import pallas as pl
from jax.experimental.pallas import tpu as pltpu
from jax.experimental.pallas import tpu_sc as plsc

VOCAB = 1000000
EMBED_DIM = 64
BATCH = 4096
HIST = 200

_INFO = plsc.get_sparse_core_info()
NC, NS, L = _INFO.num_cores, _INFO.num_subcores, _INFO.num_lanes
NW = NC * NS

B = BATCH * HIST
B_PER_W = B // NW
BATCH_PER_W = BATCH // NW
N_CHUNKS = BATCH_PER_W
NBUF = 4
LOOKAHEAD = 2


def _gather_body(x_hbm, table_hbm, out_hbm, idx_v, rows_v,
                 g0, g1, g2, g3, o0, o1, o2, o3):
    gsems = (g0, g1, g2, g3)
    osems = (o0, o1, o2, o3)
    wid = lax.axis_index("s") * NC + lax.axis_index("c")
    base = wid * B_PER_W
    b_base = wid * BATCH_PER_W
    pltpu.sync_copy(x_hbm.at[pl.ds(base, B_PER_W)], idx_v)

    def fire_gather(c, b):
        pltpu.async_copy(
            table_hbm.at[idx_v.at[pl.ds(c * HIST, HIST)]],
            rows_v.at[b], gsems[b])

    def wait_gather(c, b):
        pltpu.make_async_copy(
            table_hbm.at[idx_v.at[pl.ds(c * HIST, HIST)]],
            rows_v.at[b], gsems[b]).wait()

    def fire_out(c, b):
        pltpu.async_copy(rows_v.at[b], out_hbm.at[b_base + c], osems[b])

    def wait_out(c, b):
        pltpu.make_async_copy(
            rows_v.at[b], out_hbm.at[b_base + c], osems[b]).wait()

    for c in range(LOOKAHEAD):
        fire_gather(c, c % NBUF)

    def round_body(r, carry):
        for b in range(NBUF):
            c = r * NBUF + b
            c2 = c + LOOKAHEAD
            b2 = (b + LOOKAHEAD) % NBUF

            @pl.when(c2 < N_CHUNKS)
            def _():
                @pl.when(c2 >= NBUF)
                def _():
                    wait_out(c2 - NBUF, b2)
                fire_gather(c2, b2)

            wait_gather(c, b)
            fire_out(c, b)
        return carry

    lax.fori_loop(0, N_CHUNKS // NBUF, round_body, 0)

    for k in range(NBUF):
        c = N_CHUNKS - NBUF + k
        wait_out(c, c % NBUF)


def kernel(x, table):
    x1 = x.reshape(B).astype(jnp.int32)
    mesh = plsc.VectorSubcoreMesh(core_axis_name="c", subcore_axis_name="s")
    return pl.kernel(
        _gather_body,
        mesh=mesh,
        out_type=jax.ShapeDtypeStruct((BATCH, HIST, EMBED_DIM), jnp.float32),
        scratch_types=[
            pltpu.VMEM((B_PER_W,), jnp.int32),
            pltpu.VMEM((NBUF, HIST, EMBED_DIM), jnp.float32),
        ] + [pltpu.SemaphoreType.DMA] * (2 * NBUF),
        compiler_params=pltpu.CompilerParams(use_tc_tiling_on_sc=False),
    )(x1, table)


# speedup vs baseline: 1.3497x; 1.3263x over previous
import jax
import jax.numpy as jnp
from jax import lax
from jax.experimental import pallas as pl
from jax.experimental.pallas import tpu as pltpu
from jax.experimental.pallas import tpu_sc as plsc

VOCAB = 1000000
EMBED_DIM = 64
BATCH = 4096
HIST = 200

_INFO = plsc.get_sparse_core_info()
NC, NS, L = _INFO.num_cores, _INFO.num_subcores, _INFO.num_lanes
NW = NC * NS

B = BATCH * HIST
B_PER_W = B // NW
BATCH_PER_W = BATCH // NW
N_CHUNKS = BATCH_PER_W
NBUF = 4
LOOKAHEAD = 2


def _gather_body(x_hbm, table_hbm, out_hbm, idx_v, rows_v,
                 g0, g1, g2, g3, o0, o1, o2, o3):
    gsems = (g0, g1, g2, g3)
    osems = (o0, o1, o2, o3)
    wid = lax.axis_index("s") * NC + lax.axis_index("c")
    base = wid * B_PER_W
    b_base = wid * BATCH_PER_W
    pltpu.sync_copy(x_hbm.at[pl.ds(base, B_PER_W)], idx_v)

    def fire_gather(c, b):
        pltpu.async_copy(
            table_hbm.at[idx_v.at[pl.ds(c * HIST, HIST)]],
            rows_v.at[b], gsems[b])

    def wait_gather(c, b):
        pltpu.make_async_copy(
            table_hbm.at[idx_v.at[pl.ds(c * HIST, HIST)]],
            rows_v.at[b], gsems[b]).wait()

    def fire_out(c, b):
        pltpu.async_copy(
            rows_v.at[b],
            out_hbm.at[b_base + c, :, pl.ds(0, EMBED_DIM)], osems[b])

    def wait_out(c, b):
        pltpu.make_async_copy(
            rows_v.at[b],
            out_hbm.at[b_base + c, :, pl.ds(0, EMBED_DIM)], osems[b]).wait()

    for c in range(LOOKAHEAD):
        fire_gather(c, c % NBUF)

    def round_body(r, carry):
        for b in range(NBUF):
            c = r * NBUF + b
            c2 = c + LOOKAHEAD
            b2 = (b + LOOKAHEAD) % NBUF

            @pl.when(c2 < N_CHUNKS)
            def _():
                @pl.when(c2 >= NBUF)
                def _():
                    wait_out(c2 - NBUF, b2)
                fire_gather(c2, b2)

            wait_gather(c, b)
            fire_out(c, b)
        return carry

    lax.fori_loop(0, N_CHUNKS // NBUF, round_body, 0)

    for k in range(NBUF):
        c = N_CHUNKS - NBUF + k
        wait_out(c, c % NBUF)


def kernel(x, table):
    x1 = x.reshape(B).astype(jnp.int32)
    mesh = plsc.VectorSubcoreMesh(core_axis_name="c", subcore_axis_name="s")
    padded = pl.kernel(
        _gather_body,
        mesh=mesh,
        out_type=jax.ShapeDtypeStruct((BATCH, HIST, 2 * EMBED_DIM), jnp.float32),
        scratch_types=[
            pltpu.VMEM((B_PER_W,), jnp.int32),
            pltpu.VMEM((NBUF, HIST, EMBED_DIM), jnp.float32),
        ] + [pltpu.SemaphoreType.DMA] * (2 * NBUF),
        compiler_params=pltpu.CompilerParams(use_tc_tiling_on_sc=False),
    )(x1, table)
    return padded[:, :, :EMBED_DIM]
